# baseline (speedup 1.0000x reference)
import jax
import jax.numpy as jnp
from jax import lax
from jax.experimental import pallas as pl
from jax.experimental.pallas import tpu as pltpu

_TB = 1024
_NSTEPS = 4


def _fused_pool_fc_kernel(x_ref, w_ref, out_ref, w2_ref):
    @pl.when(pl.program_id(1) == 0)
    def _build_w2():
        w = w_ref[...]
        w2_ref[...] = jnp.stack(
            [w, jnp.zeros_like(w)], axis=-1).reshape(w.shape[0], -1)

    x = x_ref[...]
    pooled_full = jnp.maximum(x, pltpu.roll(x, x.shape[1] - 1, 1))
    out_ref[...] = lax.dot_general(
        pooled_full, w2_ref[...],
        dimension_numbers=(((1,), (1,)), ((), ())),
        preferred_element_type=jnp.float32)


def kernel(x, fc1_weight):
    Bx, C, L = x.shape
    n_classes, K = fc1_weight.shape
    xflat = x.reshape(Bx, C * L)

    tb = min(_TB, Bx)
    nouter = max(1, pl.cdiv(Bx, tb * _NSTEPS))
    nsteps = pl.cdiv(Bx, tb * nouter)
    grid = (nouter, nsteps)
    cost = pl.CostEstimate(
        flops=2 * Bx * K * n_classes + Bx * C * L,
        transcendentals=0,
        bytes_accessed=4 * (Bx * C * L + n_classes * K + Bx * n_classes),
    )
    return pl.pallas_call(
        _fused_pool_fc_kernel,
        out_shape=jax.ShapeDtypeStruct((Bx, n_classes), jnp.float32),
        grid=grid,
        in_specs=[pl.BlockSpec((tb, C * L), lambda o, i: (o * _NSTEPS + i, 0)),
                  pl.BlockSpec((n_classes, K), lambda o, i: (0, 0))],
        out_specs=pl.BlockSpec((tb, n_classes), lambda o, i: (o * _NSTEPS + i, 0)),
        scratch_shapes=[pltpu.VMEM((n_classes, C * L), jnp.float32)],
        compiler_params=pltpu.CompilerParams(
            dimension_semantics=("parallel", "arbitrary")),
        cost_estimate=cost,
    )(xflat, fc1_weight)


# speedup vs baseline: 1.0177x; 1.0177x over previous
import jax
import jax.numpy as jnp
from jax import lax
from jax.experimental import pallas as pl
from jax.experimental.pallas import tpu as pltpu

_TB = 2048
_NSTEPS = 2


def _fused_pool_fc_kernel(x_ref, w_ref, out_ref, w2_ref):
    @pl.when(pl.program_id(1) == 0)
    def _build_w2():
        w = w_ref[...]
        w2_ref[...] = jnp.stack(
            [w, jnp.zeros_like(w)], axis=-1).reshape(w.shape[0], -1)

    x = x_ref[...]
    pooled_full = jnp.maximum(x, pltpu.roll(x, x.shape[1] - 1, 1))
    out_ref[...] = lax.dot_general(
        pooled_full, w2_ref[...],
        dimension_numbers=(((1,), (1,)), ((), ())),
        preferred_element_type=jnp.float32)


def kernel(x, fc1_weight):
    Bx, C, L = x.shape
    n_classes, K = fc1_weight.shape
    xflat = x.reshape(Bx, C * L)

    tb = min(_TB, Bx)
    nouter = max(1, pl.cdiv(Bx, tb * _NSTEPS))
    nsteps = pl.cdiv(Bx, tb * nouter)
    grid = (nouter, nsteps)
    cost = pl.CostEstimate(
        flops=2 * Bx * K * n_classes + Bx * C * L,
        transcendentals=0,
        bytes_accessed=4 * (Bx * C * L + n_classes * K + Bx * n_classes),
    )
    return pl.pallas_call(
        _fused_pool_fc_kernel,
        out_shape=jax.ShapeDtypeStruct((Bx, n_classes), jnp.float32),
        grid=grid,
        in_specs=[pl.BlockSpec((tb, C * L), lambda o, i: (o * _NSTEPS + i, 0)),
                  pl.BlockSpec((n_classes, K), lambda o, i: (0, 0))],
        out_specs=pl.BlockSpec((tb, n_classes), lambda o, i: (o * _NSTEPS + i, 0)),
        scratch_shapes=[pltpu.VMEM((n_classes, C * L), jnp.float32)],
        compiler_params=pltpu.CompilerParams(
            dimension_semantics=("parallel", "arbitrary")),
        cost_estimate=cost,
    )(xflat, fc1_weight)
